# Initial kernel scaffold; baseline (speedup 1.0000x reference)
import jax
import jax.numpy as jnp
from jax.experimental import pallas as pl


def kernel(node_feat, edge_feat, edge_index, graph_ids, params):
    raise NotImplementedError("write your pallas kernel here")


# speedup vs baseline: 2.5393x; 2.5393x over previous
import functools

import jax
import jax.numpy as jnp
from jax import lax
from jax.experimental import pallas as pl
from jax.experimental.pallas import tpu as pltpu
from jax.experimental.pallas import tpu_sc as plsc

N = 10000
E = 160000
G = 16
D_NODE = 256
D_EDGE = 16
LATENT = 256
MULTI_H = 8
MAX_K = 3
MAX_BLOCK = 3

NPAD = 10240
EPAD = 163840
NSUB = 16
LANES = 16
CHUNK = 128
CPS = EPAD // NSUB // CHUNK
GROUPS = 5
GCH = CPS // GROUPS
ROWS_PER_SUB = NPAD // NSUB
HALF = 128
EPS = 1e-5


@functools.lru_cache(maxsize=None)
def _make_sc_segsum():
    mesh = plsc.VectorSubcoreMesh(core_axis_name="c", subcore_axis_name="s")
    out_t = [jax.ShapeDtypeStruct((NPAD, HALF), jnp.float32)] * 2

    @functools.partial(
        pl.kernel, mesh=mesh, out_type=out_t,
        scratch_types=[
            pltpu.VMEM((GCH, CHUNK), jnp.int32),
            pltpu.VMEM((GCH, CHUNK), jnp.int32),
            pltpu.VMEM((CHUNK, HALF), jnp.float32),
            pltpu.VMEM((CHUNK, HALF), jnp.float32),
            pltpu.VMEM_SHARED((NPAD, HALF), jnp.float32),
            pltpu.SemaphoreType.DMA,
            pltpu.SemaphoreType.DMA,
        ])
    def sc_segsum(x0, x1, isrc, idst, o0, o1,
                  isrc_v, idst_v, r_a, r_b, acc, sem_a, sem_b):
        cid = lax.axis_index("c")
        sid = lax.axis_index("s")

        @pl.loop(0, CHUNK)
        def _(r):
            for j in range(HALF // LANES):
                r_a[r, pl.ds(j * LANES, LANES)] = jnp.zeros((LANES,), jnp.float32)

        for i in range(ROWS_PER_SUB // CHUNK):
            pltpu.sync_copy(
                r_a, acc.at[pl.ds(sid * ROWS_PER_SUB + i * CHUNK, CHUNK)])
        plsc.subcore_barrier()

        def run(tab, out):
            def issue(k, buf, sem):
                pltpu.async_copy(tab.at[isrc_v.at[k]], buf, sem)

            def wait(buf, sem):
                pltpu.make_async_copy(tab.at[isrc_v.at[0]], buf, sem).wait()

            def scat(buf, k):
                pltpu.sync_copy(buf, acc.at[idst_v.at[k]], add=True)

            for g in range(GROUPS):
                pltpu.sync_copy(isrc.at[sid, pl.ds(g * GCH, GCH)], isrc_v)
                pltpu.sync_copy(idst.at[sid, pl.ds(g * GCH, GCH)], idst_v)
                issue(0, r_a, sem_a)

                @pl.loop(0, GCH - 2, step=2)
                def _(k):
                    wait(r_a, sem_a)
                    issue(k + 1, r_b, sem_b)
                    scat(r_a, k)
                    wait(r_b, sem_b)
                    issue(k + 2, r_a, sem_a)
                    scat(r_b, k + 1)

                wait(r_a, sem_a)
                issue(GCH - 1, r_b, sem_b)
                scat(r_a, GCH - 2)
                wait(r_b, sem_b)
                scat(r_b, GCH - 1)
            plsc.subcore_barrier()
            pltpu.sync_copy(acc.at[pl.ds(sid * ROWS_PER_SUB, ROWS_PER_SUB)],
                            out.at[pl.ds(sid * ROWS_PER_SUB, ROWS_PER_SUB)])

        @pl.when(cid == 0)
        def _():
            run(x0, o0)

        @pl.when(cid == 1)
        def _():
            run(x1, o1)

    return sc_segsum


def _sc_segsum_node(x0, x1, isrc, idst):
    return _make_sc_segsum()(x0, x1, isrc, idst)


def _sc_segsum_edge(x0, x1, isrc, idst):
    return _make_sc_segsum()(x0, x1, isrc, idst)


@functools.lru_cache(maxsize=None)
def _make_sc_deg():
    mesh = plsc.VectorSubcoreMesh(core_axis_name="c", subcore_axis_name="s")

    @functools.partial(
        pl.kernel, mesh=mesh,
        out_type=jax.ShapeDtypeStruct((NPAD, HALF), jnp.float32),
        scratch_types=[
            pltpu.VMEM((CPS, CHUNK), jnp.int32),
            pltpu.VMEM((CHUNK, HALF), jnp.float32),
            pltpu.VMEM_SHARED((NPAD, HALF), jnp.float32),
        ])
    def sc_deg(idst, o, idst_v, r_a, acc):
        cid = lax.axis_index("c")
        sid = lax.axis_index("s")

        @pl.when(cid == 0)
        def _():
            @pl.loop(0, CHUNK)
            def _(r):
                for j in range(HALF // LANES):
                    r_a[r, pl.ds(j * LANES, LANES)] = jnp.zeros(
                        (LANES,), jnp.float32)

            for i in range(ROWS_PER_SUB // CHUNK):
                pltpu.sync_copy(
                    r_a, acc.at[pl.ds(sid * ROWS_PER_SUB + i * CHUNK, CHUNK)])
            pltpu.sync_copy(idst.at[sid], idst_v)

            @pl.loop(0, CHUNK)
            def _(r):
                for j in range(HALF // LANES):
                    r_a[r, pl.ds(j * LANES, LANES)] = jnp.ones(
                        (LANES,), jnp.float32)

            plsc.subcore_barrier()

            @pl.loop(0, CPS)
            def _(k):
                pltpu.sync_copy(r_a, acc.at[idst_v.at[k]], add=True)

            plsc.subcore_barrier()
            pltpu.sync_copy(acc.at[pl.ds(sid * ROWS_PER_SUB, ROWS_PER_SUB)],
                            o.at[pl.ds(sid * ROWS_PER_SUB, ROWS_PER_SUB)])

    return sc_deg


def _sc_deg(idst):
    return _make_sc_deg()(idst)


_TILE = 256


def _row_mask(i, tile, n_valid):
    rid = i * tile + lax.broadcasted_iota(jnp.int32, (tile, 1), 0)
    return (rid < n_valid).astype(jnp.float32)


def _acc_stats(acc_ref, st_ref, z, i, grid, tile, n_valid):
    zm = z * _row_mask(i, tile, n_valid)

    @pl.when(i == 0)
    def _():
        acc_ref[...] = jnp.zeros_like(acc_ref)

    acc_ref[0:1, :] += jnp.sum(zm, axis=0, keepdims=True)
    acc_ref[1:2, :] += jnp.sum(zm * zm, axis=0, keepdims=True)

    @pl.when(i == grid - 1)
    def _():
        st_ref[...] = acc_ref[...]


def _tc_matmul_stats(x, w, b, n_valid, tile=_TILE):
    rows, kdim = x.shape
    cdim = w.shape[1]
    grid = rows // tile

    def body(x_ref, w_ref, b_ref, o_ref, st_ref, acc_ref):
        i = pl.program_id(0)
        z = jnp.dot(x_ref[...], w_ref[...],
                    preferred_element_type=jnp.float32) + b_ref[...]
        o_ref[...] = z
        _acc_stats(acc_ref, st_ref, z, i, grid, tile, n_valid)

    return pl.pallas_call(
        body,
        grid=(grid,),
        in_specs=[
            pl.BlockSpec((tile, kdim), lambda i: (i, 0)),
            pl.BlockSpec((kdim, cdim), lambda i: (0, 0)),
            pl.BlockSpec((1, cdim), lambda i: (0, 0)),
        ],
        out_specs=[
            pl.BlockSpec((tile, cdim), lambda i: (i, 0)),
            pl.BlockSpec((8, cdim), lambda i: (0, 0)),
        ],
        out_shape=[
            jax.ShapeDtypeStruct((rows, cdim), jnp.float32),
            jax.ShapeDtypeStruct((8, cdim), jnp.float32),
        ],
        scratch_shapes=[pltpu.VMEM((8, cdim), jnp.float32)],
    )(x, w, b)


def _tc_matmul_stats_halves(x, w, b, n_valid, tile=_TILE):
    rows, kdim = x.shape
    cdim = w.shape[1]
    grid = rows // tile

    def body(x_ref, w_ref, b_ref, o0_ref, o1_ref, st_ref, acc_ref):
        i = pl.program_id(0)
        z = jnp.dot(x_ref[...], w_ref[...],
                    preferred_element_type=jnp.float32) + b_ref[...]
        o0_ref[...] = z[:, :HALF]
        o1_ref[...] = z[:, HALF:]
        _acc_stats(acc_ref, st_ref, z, i, grid, tile, n_valid)

    return pl.pallas_call(
        body,
        grid=(grid,),
        in_specs=[
            pl.BlockSpec((tile, kdim), lambda i: (i, 0)),
            pl.BlockSpec((kdim, cdim), lambda i: (0, 0)),
            pl.BlockSpec((1, cdim), lambda i: (0, 0)),
        ],
        out_specs=[
            pl.BlockSpec((tile, HALF), lambda i: (i, 0)),
            pl.BlockSpec((tile, HALF), lambda i: (i, 0)),
            pl.BlockSpec((8, cdim), lambda i: (0, 0)),
        ],
        out_shape=[
            jax.ShapeDtypeStruct((rows, HALF), jnp.float32),
            jax.ShapeDtypeStruct((rows, HALF), jnp.float32),
            jax.ShapeDtypeStruct((8, cdim), jnp.float32),
        ],
        scratch_shapes=[pltpu.VMEM((8, cdim), jnp.float32)],
    )(x, w, b)


def _tc_input_potential(h0, s0, s1, deg, a1, c1, ae, ce):
    grid = NPAD // _TILE

    def body(h_ref, s0_ref, s1_ref, d_ref, a1_ref, c1_ref, ae_ref, ce_ref,
             o0_ref, o1_ref):
        s = jnp.concatenate([s0_ref[...], s1_ref[...]], axis=1)
        d = d_ref[:, 0:1]
        z = (a1_ref[...] * h_ref[...] + c1_ref[...]
             + ae_ref[...] * s + d * ce_ref[...])
        z = jnp.maximum(z, 0.0)
        o0_ref[...] = z[:, :HALF]
        o1_ref[...] = z[:, HALF:]

    vec = pl.BlockSpec((1, LATENT), lambda i: (0, 0))
    half = pl.BlockSpec((_TILE, HALF), lambda i: (i, 0))
    return pl.pallas_call(
        body,
        grid=(grid,),
        in_specs=[pl.BlockSpec((_TILE, LATENT), lambda i: (i, 0)),
                  half, half, half, vec, vec, vec, vec],
        out_specs=[half, half],
        out_shape=[jax.ShapeDtypeStruct((NPAD, HALF), jnp.float32)] * 2,
    )(h0, s0, s1, deg, a1, c1, ae, ce)


def _tc_step(s0, s1, bi0, bi1, deg, a, c, w, b):
    grid = NPAD // _TILE

    def body(s0_ref, s1_ref, b0_ref, b1_ref, d_ref, a_ref, c_ref, w_ref,
             bias_ref, o0_ref, o1_ref, st_ref, acc_ref):
        i = pl.program_id(0)
        s = jnp.concatenate([s0_ref[...], s1_ref[...]], axis=1)
        bi = jnp.concatenate([b0_ref[...], b1_ref[...]], axis=1)
        d = d_ref[:, 0:1]
        t = a_ref[...] * s + d * c_ref[...] + bi
        z = jnp.dot(t, w_ref[...],
                    preferred_element_type=jnp.float32) + bias_ref[...]
        o0_ref[...] = z[:, :HALF]
        o1_ref[...] = z[:, HALF:]
        _acc_stats(acc_ref, st_ref, z, i, grid, _TILE, N)

    vec = pl.BlockSpec((1, LATENT), lambda i: (0, 0))
    half = pl.BlockSpec((_TILE, HALF), lambda i: (i, 0))
    return pl.pallas_call(
        body,
        grid=(grid,),
        in_specs=[half, half, half, half, half, vec, vec,
                  pl.BlockSpec((LATENT, LATENT), lambda i: (0, 0)), vec],
        out_specs=[half, half, pl.BlockSpec((8, LATENT), lambda i: (0, 0))],
        out_shape=[
            jax.ShapeDtypeStruct((NPAD, HALF), jnp.float32),
            jax.ShapeDtypeStruct((NPAD, HALF), jnp.float32),
            jax.ShapeDtypeStruct((8, LATENT), jnp.float32),
        ],
        scratch_shapes=[pltpu.VMEM((8, LATENT), jnp.float32)],
    )(s0, s1, bi0, bi1, deg, a, c, w, b)


def _tc_concat_matmul(zs, affs, ws):
    grid = NPAD // _TILE

    def body(z00, z01, z10, z11, z20, z21, a0, c0, a1, c1, a2, c2,
             w0, w1, w2, o_ref, st_ref, acc_ref):
        i = pl.program_id(0)
        halves = [(z00, z01), (z10, z11), (z20, z21)]
        acs = [(a0, c0), (a1, c1), (a2, c2)]
        wks = [w0, w1, w2]
        r = None
        for (zl, zh), (ak, ck), wk in zip(halves, acs, wks):
            z = jnp.concatenate([zl[...], zh[...]], axis=1)
            t = ak[...] * z + ck[...]
            d = jnp.dot(t, wk[...], preferred_element_type=jnp.float32)
            r = d if r is None else r + d
        o_ref[...] = r
        _acc_stats(acc_ref, st_ref, r, i, grid, _TILE, N)

    vec = pl.BlockSpec((1, LATENT), lambda i: (0, 0))
    half = pl.BlockSpec((_TILE, HALF), lambda i: (i, 0))
    wspec = pl.BlockSpec((LATENT, LATENT), lambda i: (0, 0))
    args = []
    for k in range(MAX_K):
        args += [zs[k][0], zs[k][1]]
    for k in range(MAX_K):
        args += [affs[k][0], affs[k][1]]
    args += list(ws)
    return pl.pallas_call(
        body,
        grid=(grid,),
        in_specs=[half] * 6 + [vec] * 6 + [wspec] * 3,
        out_specs=[pl.BlockSpec((_TILE, LATENT), lambda i: (i, 0)),
                   pl.BlockSpec((8, LATENT), lambda i: (0, 0))],
        out_shape=[
            jax.ShapeDtypeStruct((NPAD, LATENT), jnp.float32),
            jax.ShapeDtypeStruct((8, LATENT), jnp.float32),
        ],
        scratch_shapes=[pltpu.VMEM((8, LATENT), jnp.float32)],
    )(*args)


def _tc_block_input(r, a3, c3, ip0, ip1):
    grid = NPAD // _TILE

    def body(r_ref, a_ref, c_ref, i0_ref, i1_ref, o0_ref, o1_ref):
        cur = jnp.maximum(a_ref[...] * r_ref[...] + c_ref[...], 0.0)
        ip = jnp.concatenate([i0_ref[...], i1_ref[...]], axis=1)
        z = cur + ip
        o0_ref[...] = z[:, :HALF]
        o1_ref[...] = z[:, HALF:]

    vec = pl.BlockSpec((1, LATENT), lambda i: (0, 0))
    half = pl.BlockSpec((_TILE, HALF), lambda i: (i, 0))
    return pl.pallas_call(
        body,
        grid=(grid,),
        in_specs=[pl.BlockSpec((_TILE, LATENT), lambda i: (i, 0)),
                  vec, vec, half, half],
        out_specs=[half, half],
        out_shape=[jax.ShapeDtypeStruct((NPAD, HALF), jnp.float32)] * 2,
    )(r, a3, c3, ip0, ip1)


def _tc_node_emb(r, a3, c3, w, b):
    grid = NPAD // _TILE

    def body(r_ref, a_ref, c_ref, w_ref, b_ref, o_ref):
        cur = jnp.maximum(a_ref[...] * r_ref[...] + c_ref[...], 0.0)
        z = jnp.dot(cur, w_ref[...],
                    preferred_element_type=jnp.float32) + b_ref[...]
        o_ref[...] = jnp.maximum(z, 0.0)

    vec = pl.BlockSpec((1, LATENT), lambda i: (0, 0))
    return pl.pallas_call(
        body,
        grid=(grid,),
        in_specs=[pl.BlockSpec((_TILE, LATENT), lambda i: (i, 0)),
                  vec, vec,
                  pl.BlockSpec((LATENT, LATENT), lambda i: (0, 0)), vec],
        out_specs=pl.BlockSpec((_TILE, LATENT), lambda i: (i, 0)),
        out_shape=jax.ShapeDtypeStruct((NPAD, LATENT), jnp.float32),
    )(r, a3, c3, w, b)


def _tc_tanh_matmul_stats(x, w, b, n_valid):
    rows, kdim = x.shape
    cdim = w.shape[1]
    grid = rows // _TILE

    def body(x_ref, w_ref, b_ref, o_ref, st_ref, acc_ref):
        i = pl.program_id(0)
        z = jnp.tanh(jnp.dot(x_ref[...], w_ref[...],
                             preferred_element_type=jnp.float32) + b_ref[...])
        o_ref[...] = z
        _acc_stats(acc_ref, st_ref, z, i, grid, _TILE, n_valid)

    return pl.pallas_call(
        body,
        grid=(grid,),
        in_specs=[
            pl.BlockSpec((_TILE, kdim), lambda i: (i, 0)),
            pl.BlockSpec((kdim, cdim), lambda i: (0, 0)),
            pl.BlockSpec((1, cdim), lambda i: (0, 0)),
        ],
        out_specs=[
            pl.BlockSpec((_TILE, cdim), lambda i: (i, 0)),
            pl.BlockSpec((8, cdim), lambda i: (0, 0)),
        ],
        out_shape=[
            jax.ShapeDtypeStruct((rows, cdim), jnp.float32),
            jax.ShapeDtypeStruct((8, cdim), jnp.float32),
        ],
        scratch_shapes=[pltpu.VMEM((8, cdim), jnp.float32)],
    )(x, w, b)


def _tc_pool(a2, a5, c5, g1h, emb):
    grid = NPAD // _TILE
    gh = G * MULTI_H

    def body(a2_ref, a5_ref, c5_ref, g_ref, e_ref, o_ref, acc_ref):
        i = pl.program_id(0)
        z = a5_ref[...] * a2_ref[...] + c5_ref[...]
        z = z[:, :MULTI_H]
        z = z - jnp.max(z, axis=1, keepdims=True)
        ez = jnp.exp(z)
        alpha = ez / jnp.sum(ez, axis=1, keepdims=True)
        wt = (g_ref[...][:, :, None] * alpha[:, None, :]).reshape(_TILE, gh)
        part = lax.dot_general(wt, e_ref[...], (((0,), (0,)), ((), ())),
                               preferred_element_type=jnp.float32)

        @pl.when(i == 0)
        def _():
            acc_ref[...] = jnp.zeros_like(acc_ref)

        acc_ref[...] += part

        @pl.when(i == grid - 1)
        def _():
            o_ref[...] = jnp.maximum(acc_ref[...], 0.0)

    vec = pl.BlockSpec((1, HALF), lambda i: (0, 0))
    return pl.pallas_call(
        body,
        grid=(grid,),
        in_specs=[pl.BlockSpec((_TILE, HALF), lambda i: (i, 0)),
                  vec, vec,
                  pl.BlockSpec((_TILE, G), lambda i: (i, 0)),
                  pl.BlockSpec((_TILE, LATENT), lambda i: (i, 0))],
        out_specs=pl.BlockSpec((gh, LATENT), lambda i: (0, 0)),
        out_shape=jax.ShapeDtypeStruct((gh, LATENT), jnp.float32),
        scratch_shapes=[pltpu.VMEM((gh, LATENT), jnp.float32)],
    )(a2, a5, c5, g1h, emb)


def _tc_l1(flat):
    rows = flat.shape[0]

    def body(x_ref, o_ref):
        o_ref[...] = jnp.full(
            (1, 1), jnp.sum(jnp.abs(x_ref[...])) / G, jnp.float32)

    return pl.pallas_call(
        body,
        grid=(1,),
        in_specs=[pl.BlockSpec((rows, HALF), lambda i: (0, 0))],
        out_specs=pl.BlockSpec((1, 1), lambda i: (0, 0)),
        out_shape=jax.ShapeDtypeStruct((1, 1), jnp.float32),
    )(flat)


def _affine(st, g, b, n):
    mu = st[0] / n
    var = st[1] / n - mu * mu
    a = g / jnp.sqrt(var + EPS)
    c = b - a * mu
    return a.reshape(1, -1), c.reshape(1, -1)


def kernel(node_feat, edge_feat, edge_index, graph_ids, params):
    p = params
    i32 = jnp.int32

    src = edge_index[0]
    dst = edge_index[1]
    src_p = jnp.concatenate([src, jnp.zeros(EPAD - E, i32)]).reshape(
        NSUB, CPS, CHUNK)
    dst_p = jnp.concatenate([dst, jnp.full(EPAD - E, N, i32)]).reshape(
        NSUB, CPS, CHUNK)
    eidx_p = jnp.concatenate([jnp.arange(E, dtype=i32),
                              jnp.zeros(EPAD - E, i32)]).reshape(
        NSUB, CPS, CHUNK)

    nf = jnp.pad(node_feat, ((0, NPAD - N), (0, 0)))
    ef = jnp.pad(edge_feat, ((0, EPAD - E), (0, HALF - D_EDGE)))
    we = jnp.pad(p['w_e2l_W'], ((0, HALF - D_EDGE), (0, 0)))

    deg = _sc_deg(dst_p)

    h0, st1 = _tc_matmul_stats(nf, p['w_n2l_W'],
                               p['w_n2l_b'].reshape(1, -1), N)
    a1, c1 = _affine(st1, p['bn1_g'], p['bn1_b'], N)
    el0, el1, ste = _tc_matmul_stats_halves(ef, we,
                                            p['w_e2l_b'].reshape(1, -1), E,
                                            tile=2048)
    ae, ce = _affine(ste, p['bne1_g'], p['bne1_b'], E)

    s0, s1 = _sc_segsum_edge(el0, el1, eidx_p, dst_p)
    ip0, ip1 = _tc_input_potential(h0, s0, s1, deg, a1, c1, ae, ce)

    ones = jnp.ones((1, LATENT), jnp.float32)
    zeros = jnp.zeros((1, LATENT), jnp.float32)
    kw = [p['k_weight'][k * LATENT:(k + 1) * LATENT] for k in range(MAX_K)]

    bi0, bi1 = ip0, ip1
    r_raw = None
    a3 = c3 = None
    for block in range(MAX_BLOCK):
        tab0, tab1, a, c = bi0, bi1, ones, zeros
        zs, affs = [], []
        for step in range(MAX_K):
            sg0, sg1 = _sc_segsum_node(tab0, tab1, src_p, dst_p)
            z0, z1, stz = _tc_step(sg0, sg1, bi0, bi1, deg, a, c,
                                   p['conv_W'][step],
                                   p['conv_b'][step].reshape(1, -1))
            a, c = _affine(stz, p['bn2_g'][step], p['bn2_b'][step], N)
            tab0, tab1 = z0, z1
            zs.append((z0, z1))
            affs.append((a, c))
        r_raw, st3 = _tc_concat_matmul(zs, affs, kw)
        a3, c3 = _affine(st3, p['bn3_g'], p['bn3_b'], N)
        if block < MAX_BLOCK - 1:
            bi0, bi1 = _tc_block_input(r_raw, a3, c3, ip0, ip1)

    emb = _tc_node_emb(r_raw, a3, c3, p['out_W'], p['out_b'].reshape(1, -1))
    t_raw, st4 = _tc_tanh_matmul_stats(emb, p['att_w1_W'],
                                       p['att_w1_b'].reshape(1, -1), N)
    a4, c4 = _affine(st4, p['bn4_g'], p['bn4_b'], N)
    w2f = jnp.pad(a4.reshape(-1, 1) * p['att_w2_W'],
                  ((0, 0), (0, HALF - MULTI_H)))
    b2f = jnp.pad((c4.reshape(-1) @ p['att_w2_W'] + p['att_w2_b']).reshape(1, -1),
                  ((0, 0), (0, HALF - MULTI_H)))
    a2_raw, st5 = _tc_matmul_stats(t_raw, w2f, b2f, N)
    a5, c5 = _affine(st5[:, :MULTI_H], p['bn5_g'], p['bn5_b'], N)
    a5 = jnp.pad(a5, ((0, 0), (0, HALF - MULTI_H)))
    c5 = jnp.pad(c5, ((0, 0), (0, HALF - MULTI_H)))

    g1h = (graph_ids[:, None] == jnp.arange(G)[None, :]).astype(jnp.float32)
    g1h = jnp.pad(g1h, ((0, NPAD - N), (0, 0)))
    gh = _tc_pool(a2_raw, a5, c5, g1h, emb)
    h = gh.reshape(G, MULTI_H * LATENT)

    flat = jnp.concatenate([
        p['w_n2l_W'].reshape(-1), p['w_e2l_W'].reshape(-1),
        p['conv_W'].reshape(-1), p['k_weight'].reshape(-1),
        p['out_W'].reshape(-1), p['att_w1_W'].reshape(-1),
        p['att_w2_W'].reshape(-1)]).reshape(-1, HALF)
    reg = _tc_l1(flat).reshape(())

    return (h, reg)


# speedup vs baseline: 2.7085x; 1.0666x over previous
import functools

import jax
import jax.numpy as jnp
from jax import lax
from jax.experimental import pallas as pl
from jax.experimental.pallas import tpu as pltpu
from jax.experimental.pallas import tpu_sc as plsc

N = 10000
E = 160000
G = 16
D_NODE = 256
D_EDGE = 16
LATENT = 256
MULTI_H = 8
MAX_K = 3
MAX_BLOCK = 3

NPAD = 10240
EPAD = 163840
NSUB = 16
LANES = 16
CHUNK = 128
CPS = EPAD // NSUB // CHUNK
GROUPS = 2
GCH = CPS // GROUPS
ROWS_PER_SUB = NPAD // NSUB
HALF = 128
EPS = 1e-5


@functools.lru_cache(maxsize=None)
def _make_sc_segsum():
    mesh = plsc.VectorSubcoreMesh(core_axis_name="c", subcore_axis_name="s")
    out_t = [jax.ShapeDtypeStruct((NPAD, HALF), jnp.float32)] * 2

    @functools.partial(
        pl.kernel, mesh=mesh, out_type=out_t,
        scratch_types=[
            pltpu.VMEM((GCH, CHUNK), jnp.int32),
            pltpu.VMEM((GCH, CHUNK), jnp.int32),
            pltpu.VMEM((CHUNK, HALF), jnp.float32),
            pltpu.VMEM((CHUNK, HALF), jnp.float32),
            pltpu.VMEM_SHARED((NPAD, HALF), jnp.float32),
            pltpu.SemaphoreType.DMA,
            pltpu.SemaphoreType.DMA,
        ])
    def sc_segsum(x0, x1, isrc, idst, o0, o1,
                  isrc_v, idst_v, r_a, r_b, acc, sem_a, sem_b):
        cid = lax.axis_index("c")
        sid = lax.axis_index("s")

        @pl.loop(0, CHUNK)
        def _(r):
            for j in range(HALF // LANES):
                r_a[r, pl.ds(j * LANES, LANES)] = jnp.zeros((LANES,), jnp.float32)

        for i in range(ROWS_PER_SUB // CHUNK):
            pltpu.sync_copy(
                r_a, acc.at[pl.ds(sid * ROWS_PER_SUB + i * CHUNK, CHUNK)])
        plsc.subcore_barrier()

        def run(tab, out):
            def issue(k, buf, sem):
                pltpu.async_copy(tab.at[isrc_v.at[k]], buf, sem)

            def wait(buf, sem):
                pltpu.make_async_copy(tab.at[isrc_v.at[0]], buf, sem).wait()

            def scat(buf, k):
                pltpu.sync_copy(buf, acc.at[idst_v.at[k]], add=True)

            for g in range(GROUPS):
                pltpu.sync_copy(isrc.at[sid, pl.ds(g * GCH, GCH)], isrc_v)
                pltpu.sync_copy(idst.at[sid, pl.ds(g * GCH, GCH)], idst_v)
                issue(0, r_a, sem_a)
                issue(1, r_b, sem_b)

                @pl.loop(0, GCH - 2, step=2)
                def _(k):
                    wait(r_a, sem_a)
                    scat(r_a, k)
                    issue(k + 2, r_a, sem_a)
                    wait(r_b, sem_b)
                    scat(r_b, k + 1)
                    issue(k + 3, r_b, sem_b)

                wait(r_a, sem_a)
                scat(r_a, GCH - 2)
                wait(r_b, sem_b)
                scat(r_b, GCH - 1)
            plsc.subcore_barrier()
            pltpu.sync_copy(acc.at[pl.ds(sid * ROWS_PER_SUB, ROWS_PER_SUB)],
                            out.at[pl.ds(sid * ROWS_PER_SUB, ROWS_PER_SUB)])

        @pl.when(cid == 0)
        def _():
            run(x0, o0)

        @pl.when(cid == 1)
        def _():
            run(x1, o1)

    return sc_segsum


def _sc_segsum_node(x0, x1, isrc, idst):
    return _make_sc_segsum()(x0, x1, isrc, idst)


def _sc_segsum_edge(x0, x1, isrc, idst):
    return _make_sc_segsum()(x0, x1, isrc, idst)


@functools.lru_cache(maxsize=None)
def _make_sc_deg():
    mesh = plsc.VectorSubcoreMesh(core_axis_name="c", subcore_axis_name="s")

    @functools.partial(
        pl.kernel, mesh=mesh,
        out_type=jax.ShapeDtypeStruct((NPAD, HALF), jnp.float32),
        scratch_types=[
            pltpu.VMEM((CPS, CHUNK), jnp.int32),
            pltpu.VMEM((CHUNK, HALF), jnp.float32),
            pltpu.VMEM_SHARED((NPAD, HALF), jnp.float32),
        ])
    def sc_deg(idst, o, idst_v, r_a, acc):
        cid = lax.axis_index("c")
        sid = lax.axis_index("s")

        @pl.when(cid == 0)
        def _():
            @pl.loop(0, CHUNK)
            def _(r):
                for j in range(HALF // LANES):
                    r_a[r, pl.ds(j * LANES, LANES)] = jnp.zeros(
                        (LANES,), jnp.float32)

            for i in range(ROWS_PER_SUB // CHUNK):
                pltpu.sync_copy(
                    r_a, acc.at[pl.ds(sid * ROWS_PER_SUB + i * CHUNK, CHUNK)])
            pltpu.sync_copy(idst.at[sid], idst_v)

            @pl.loop(0, CHUNK)
            def _(r):
                for j in range(HALF // LANES):
                    r_a[r, pl.ds(j * LANES, LANES)] = jnp.ones(
                        (LANES,), jnp.float32)

            plsc.subcore_barrier()

            @pl.loop(0, CPS)
            def _(k):
                pltpu.sync_copy(r_a, acc.at[idst_v.at[k]], add=True)

            plsc.subcore_barrier()
            pltpu.sync_copy(acc.at[pl.ds(sid * ROWS_PER_SUB, ROWS_PER_SUB)],
                            o.at[pl.ds(sid * ROWS_PER_SUB, ROWS_PER_SUB)])

    return sc_deg


def _sc_deg(idst):
    return _make_sc_deg()(idst)


_TILE = 256


def _row_mask(i, tile, n_valid):
    rid = i * tile + lax.broadcasted_iota(jnp.int32, (tile, 1), 0)
    return (rid < n_valid).astype(jnp.float32)


def _acc_stats(acc_ref, st_ref, z, i, grid, tile, n_valid):
    zm = z * _row_mask(i, tile, n_valid)

    @pl.when(i == 0)
    def _():
        acc_ref[...] = jnp.zeros_like(acc_ref)

    acc_ref[0:1, :] += jnp.sum(zm, axis=0, keepdims=True)
    acc_ref[1:2, :] += jnp.sum(zm * zm, axis=0, keepdims=True)

    @pl.when(i == grid - 1)
    def _():
        st_ref[...] = acc_ref[...]


def _tc_matmul_stats(x, w, b, n_valid, tile=_TILE):
    rows, kdim = x.shape
    cdim = w.shape[1]
    grid = rows // tile

    def body(x_ref, w_ref, b_ref, o_ref, st_ref, acc_ref):
        i = pl.program_id(0)
        z = jnp.dot(x_ref[...], w_ref[...],
                    preferred_element_type=jnp.float32) + b_ref[...]
        o_ref[...] = z
        _acc_stats(acc_ref, st_ref, z, i, grid, tile, n_valid)

    return pl.pallas_call(
        body,
        grid=(grid,),
        in_specs=[
            pl.BlockSpec((tile, kdim), lambda i: (i, 0)),
            pl.BlockSpec((kdim, cdim), lambda i: (0, 0)),
            pl.BlockSpec((1, cdim), lambda i: (0, 0)),
        ],
        out_specs=[
            pl.BlockSpec((tile, cdim), lambda i: (i, 0)),
            pl.BlockSpec((8, cdim), lambda i: (0, 0)),
        ],
        out_shape=[
            jax.ShapeDtypeStruct((rows, cdim), jnp.float32),
            jax.ShapeDtypeStruct((8, cdim), jnp.float32),
        ],
        scratch_shapes=[pltpu.VMEM((8, cdim), jnp.float32)],
    )(x, w, b)


def _tc_matmul_stats_halves(x, w, b, n_valid, tile=_TILE):
    rows, kdim = x.shape
    cdim = w.shape[1]
    grid = rows // tile

    def body(x_ref, w_ref, b_ref, o0_ref, o1_ref, st_ref, acc_ref):
        i = pl.program_id(0)
        z = jnp.dot(x_ref[...], w_ref[...],
                    preferred_element_type=jnp.float32) + b_ref[...]
        o0_ref[...] = z[:, :HALF]
        o1_ref[...] = z[:, HALF:]
        _acc_stats(acc_ref, st_ref, z, i, grid, tile, n_valid)

    return pl.pallas_call(
        body,
        grid=(grid,),
        in_specs=[
            pl.BlockSpec((tile, kdim), lambda i: (i, 0)),
            pl.BlockSpec((kdim, cdim), lambda i: (0, 0)),
            pl.BlockSpec((1, cdim), lambda i: (0, 0)),
        ],
        out_specs=[
            pl.BlockSpec((tile, HALF), lambda i: (i, 0)),
            pl.BlockSpec((tile, HALF), lambda i: (i, 0)),
            pl.BlockSpec((8, cdim), lambda i: (0, 0)),
        ],
        out_shape=[
            jax.ShapeDtypeStruct((rows, HALF), jnp.float32),
            jax.ShapeDtypeStruct((rows, HALF), jnp.float32),
            jax.ShapeDtypeStruct((8, cdim), jnp.float32),
        ],
        scratch_shapes=[pltpu.VMEM((8, cdim), jnp.float32)],
    )(x, w, b)


def _tc_input_potential(h0, s0, s1, deg, a1, c1, ae, ce):
    grid = NPAD // _TILE

    def body(h_ref, s0_ref, s1_ref, d_ref, a1_ref, c1_ref, ae_ref, ce_ref,
             o0_ref, o1_ref):
        s = jnp.concatenate([s0_ref[...], s1_ref[...]], axis=1)
        d = d_ref[:, 0:1]
        z = (a1_ref[...] * h_ref[...] + c1_ref[...]
             + ae_ref[...] * s + d * ce_ref[...])
        z = jnp.maximum(z, 0.0)
        o0_ref[...] = z[:, :HALF]
        o1_ref[...] = z[:, HALF:]

    vec = pl.BlockSpec((1, LATENT), lambda i: (0, 0))
    half = pl.BlockSpec((_TILE, HALF), lambda i: (i, 0))
    return pl.pallas_call(
        body,
        grid=(grid,),
        in_specs=[pl.BlockSpec((_TILE, LATENT), lambda i: (i, 0)),
                  half, half, half, vec, vec, vec, vec],
        out_specs=[half, half],
        out_shape=[jax.ShapeDtypeStruct((NPAD, HALF), jnp.float32)] * 2,
    )(h0, s0, s1, deg, a1, c1, ae, ce)


def _tc_step(s0, s1, bi0, bi1, deg, a, c, w, b):
    grid = NPAD // _TILE

    def body(s0_ref, s1_ref, b0_ref, b1_ref, d_ref, a_ref, c_ref, w_ref,
             bias_ref, o0_ref, o1_ref, st_ref, acc_ref):
        i = pl.program_id(0)
        s = jnp.concatenate([s0_ref[...], s1_ref[...]], axis=1)
        bi = jnp.concatenate([b0_ref[...], b1_ref[...]], axis=1)
        d = d_ref[:, 0:1]
        t = a_ref[...] * s + d * c_ref[...] + bi
        z = jnp.dot(t, w_ref[...],
                    preferred_element_type=jnp.float32) + bias_ref[...]
        o0_ref[...] = z[:, :HALF]
        o1_ref[...] = z[:, HALF:]
        _acc_stats(acc_ref, st_ref, z, i, grid, _TILE, N)

    vec = pl.BlockSpec((1, LATENT), lambda i: (0, 0))
    half = pl.BlockSpec((_TILE, HALF), lambda i: (i, 0))
    return pl.pallas_call(
        body,
        grid=(grid,),
        in_specs=[half, half, half, half, half, vec, vec,
                  pl.BlockSpec((LATENT, LATENT), lambda i: (0, 0)), vec],
        out_specs=[half, half, pl.BlockSpec((8, LATENT), lambda i: (0, 0))],
        out_shape=[
            jax.ShapeDtypeStruct((NPAD, HALF), jnp.float32),
            jax.ShapeDtypeStruct((NPAD, HALF), jnp.float32),
            jax.ShapeDtypeStruct((8, LATENT), jnp.float32),
        ],
        scratch_shapes=[pltpu.VMEM((8, LATENT), jnp.float32)],
    )(s0, s1, bi0, bi1, deg, a, c, w, b)


def _tc_concat_matmul(zs, affs, ws):
    grid = NPAD // _TILE

    def body(z00, z01, z10, z11, z20, z21, a0, c0, a1, c1, a2, c2,
             w0, w1, w2, o_ref, st_ref, acc_ref):
        i = pl.program_id(0)
        halves = [(z00, z01), (z10, z11), (z20, z21)]
        acs = [(a0, c0), (a1, c1), (a2, c2)]
        wks = [w0, w1, w2]
        r = None
        for (zl, zh), (ak, ck), wk in zip(halves, acs, wks):
            z = jnp.concatenate([zl[...], zh[...]], axis=1)
            t = ak[...] * z + ck[...]
            d = jnp.dot(t, wk[...], preferred_element_type=jnp.float32)
            r = d if r is None else r + d
        o_ref[...] = r
        _acc_stats(acc_ref, st_ref, r, i, grid, _TILE, N)

    vec = pl.BlockSpec((1, LATENT), lambda i: (0, 0))
    half = pl.BlockSpec((_TILE, HALF), lambda i: (i, 0))
    wspec = pl.BlockSpec((LATENT, LATENT), lambda i: (0, 0))
    args = []
    for k in range(MAX_K):
        args += [zs[k][0], zs[k][1]]
    for k in range(MAX_K):
        args += [affs[k][0], affs[k][1]]
    args += list(ws)
    return pl.pallas_call(
        body,
        grid=(grid,),
        in_specs=[half] * 6 + [vec] * 6 + [wspec] * 3,
        out_specs=[pl.BlockSpec((_TILE, LATENT), lambda i: (i, 0)),
                   pl.BlockSpec((8, LATENT), lambda i: (0, 0))],
        out_shape=[
            jax.ShapeDtypeStruct((NPAD, LATENT), jnp.float32),
            jax.ShapeDtypeStruct((8, LATENT), jnp.float32),
        ],
        scratch_shapes=[pltpu.VMEM((8, LATENT), jnp.float32)],
    )(*args)


def _tc_block_input(r, a3, c3, ip0, ip1):
    grid = NPAD // _TILE

    def body(r_ref, a_ref, c_ref, i0_ref, i1_ref, o0_ref, o1_ref):
        cur = jnp.maximum(a_ref[...] * r_ref[...] + c_ref[...], 0.0)
        ip = jnp.concatenate([i0_ref[...], i1_ref[...]], axis=1)
        z = cur + ip
        o0_ref[...] = z[:, :HALF]
        o1_ref[...] = z[:, HALF:]

    vec = pl.BlockSpec((1, LATENT), lambda i: (0, 0))
    half = pl.BlockSpec((_TILE, HALF), lambda i: (i, 0))
    return pl.pallas_call(
        body,
        grid=(grid,),
        in_specs=[pl.BlockSpec((_TILE, LATENT), lambda i: (i, 0)),
                  vec, vec, half, half],
        out_specs=[half, half],
        out_shape=[jax.ShapeDtypeStruct((NPAD, HALF), jnp.float32)] * 2,
    )(r, a3, c3, ip0, ip1)


def _tc_node_emb(r, a3, c3, w, b):
    grid = NPAD // _TILE

    def body(r_ref, a_ref, c_ref, w_ref, b_ref, o_ref):
        cur = jnp.maximum(a_ref[...] * r_ref[...] + c_ref[...], 0.0)
        z = jnp.dot(cur, w_ref[...],
                    preferred_element_type=jnp.float32) + b_ref[...]
        o_ref[...] = jnp.maximum(z, 0.0)

    vec = pl.BlockSpec((1, LATENT), lambda i: (0, 0))
    return pl.pallas_call(
        body,
        grid=(grid,),
        in_specs=[pl.BlockSpec((_TILE, LATENT), lambda i: (i, 0)),
                  vec, vec,
                  pl.BlockSpec((LATENT, LATENT), lambda i: (0, 0)), vec],
        out_specs=pl.BlockSpec((_TILE, LATENT), lambda i: (i, 0)),
        out_shape=jax.ShapeDtypeStruct((NPAD, LATENT), jnp.float32),
    )(r, a3, c3, w, b)


def _tc_tanh_matmul_stats(x, w, b, n_valid):
    rows, kdim = x.shape
    cdim = w.shape[1]
    grid = rows // _TILE

    def body(x_ref, w_ref, b_ref, o_ref, st_ref, acc_ref):
        i = pl.program_id(0)
        z = jnp.tanh(jnp.dot(x_ref[...], w_ref[...],
                             preferred_element_type=jnp.float32) + b_ref[...])
        o_ref[...] = z
        _acc_stats(acc_ref, st_ref, z, i, grid, _TILE, n_valid)

    return pl.pallas_call(
        body,
        grid=(grid,),
        in_specs=[
            pl.BlockSpec((_TILE, kdim), lambda i: (i, 0)),
            pl.BlockSpec((kdim, cdim), lambda i: (0, 0)),
            pl.BlockSpec((1, cdim), lambda i: (0, 0)),
        ],
        out_specs=[
            pl.BlockSpec((_TILE, cdim), lambda i: (i, 0)),
            pl.BlockSpec((8, cdim), lambda i: (0, 0)),
        ],
        out_shape=[
            jax.ShapeDtypeStruct((rows, cdim), jnp.float32),
            jax.ShapeDtypeStruct((8, cdim), jnp.float32),
        ],
        scratch_shapes=[pltpu.VMEM((8, cdim), jnp.float32)],
    )(x, w, b)


def _tc_pool(a2, a5, c5, g1h, emb):
    grid = NPAD // _TILE
    gh = G * MULTI_H

    def body(a2_ref, a5_ref, c5_ref, g_ref, e_ref, o_ref, acc_ref):
        i = pl.program_id(0)
        z = a5_ref[...] * a2_ref[...] + c5_ref[...]
        z = z[:, :MULTI_H]
        z = z - jnp.max(z, axis=1, keepdims=True)
        ez = jnp.exp(z)
        alpha = ez / jnp.sum(ez, axis=1, keepdims=True)
        wt = (g_ref[...][:, :, None] * alpha[:, None, :]).reshape(_TILE, gh)
        part = lax.dot_general(wt, e_ref[...], (((0,), (0,)), ((), ())),
                               preferred_element_type=jnp.float32)

        @pl.when(i == 0)
        def _():
            acc_ref[...] = jnp.zeros_like(acc_ref)

        acc_ref[...] += part

        @pl.when(i == grid - 1)
        def _():
            o_ref[...] = jnp.maximum(acc_ref[...], 0.0)

    vec = pl.BlockSpec((1, HALF), lambda i: (0, 0))
    return pl.pallas_call(
        body,
        grid=(grid,),
        in_specs=[pl.BlockSpec((_TILE, HALF), lambda i: (i, 0)),
                  vec, vec,
                  pl.BlockSpec((_TILE, G), lambda i: (i, 0)),
                  pl.BlockSpec((_TILE, LATENT), lambda i: (i, 0))],
        out_specs=pl.BlockSpec((gh, LATENT), lambda i: (0, 0)),
        out_shape=jax.ShapeDtypeStruct((gh, LATENT), jnp.float32),
        scratch_shapes=[pltpu.VMEM((gh, LATENT), jnp.float32)],
    )(a2, a5, c5, g1h, emb)


def _tc_l1(flat):
    rows = flat.shape[0]

    def body(x_ref, o_ref):
        o_ref[...] = jnp.full(
            (1, 1), jnp.sum(jnp.abs(x_ref[...])) / G, jnp.float32)

    return pl.pallas_call(
        body,
        grid=(1,),
        in_specs=[pl.BlockSpec((rows, HALF), lambda i: (0, 0))],
        out_specs=pl.BlockSpec((1, 1), lambda i: (0, 0)),
        out_shape=jax.ShapeDtypeStruct((1, 1), jnp.float32),
    )(flat)


def _affine(st, g, b, n):
    mu = st[0] / n
    var = st[1] / n - mu * mu
    a = g / jnp.sqrt(var + EPS)
    c = b - a * mu
    return a.reshape(1, -1), c.reshape(1, -1)


def kernel(node_feat, edge_feat, edge_index, graph_ids, params):
    p = params
    i32 = jnp.int32

    src = edge_index[0]
    dst = edge_index[1]
    src_p = jnp.concatenate([src, jnp.zeros(EPAD - E, i32)]).reshape(
        NSUB, CPS, CHUNK)
    dst_p = jnp.concatenate([dst, jnp.full(EPAD - E, N, i32)]).reshape(
        NSUB, CPS, CHUNK)
    eidx_p = jnp.concatenate([jnp.arange(E, dtype=i32),
                              jnp.zeros(EPAD - E, i32)]).reshape(
        NSUB, CPS, CHUNK)

    nf = jnp.pad(node_feat, ((0, NPAD - N), (0, 0)))
    ef = jnp.pad(edge_feat, ((0, EPAD - E), (0, HALF - D_EDGE)))
    we = jnp.pad(p['w_e2l_W'], ((0, HALF - D_EDGE), (0, 0)))

    deg = _sc_deg(dst_p)

    h0, st1 = _tc_matmul_stats(nf, p['w_n2l_W'],
                               p['w_n2l_b'].reshape(1, -1), N)
    a1, c1 = _affine(st1, p['bn1_g'], p['bn1_b'], N)
    el0, el1, ste = _tc_matmul_stats_halves(ef, we,
                                            p['w_e2l_b'].reshape(1, -1), E,
                                            tile=2048)
    ae, ce = _affine(ste, p['bne1_g'], p['bne1_b'], E)

    s0, s1 = _sc_segsum_edge(el0, el1, eidx_p, dst_p)
    ip0, ip1 = _tc_input_potential(h0, s0, s1, deg, a1, c1, ae, ce)

    ones = jnp.ones((1, LATENT), jnp.float32)
    zeros = jnp.zeros((1, LATENT), jnp.float32)
    kw = [p['k_weight'][k * LATENT:(k + 1) * LATENT] for k in range(MAX_K)]

    bi0, bi1 = ip0, ip1
    r_raw = None
    a3 = c3 = None
    for block in range(MAX_BLOCK):
        tab0, tab1, a, c = bi0, bi1, ones, zeros
        zs, affs = [], []
        for step in range(MAX_K):
            sg0, sg1 = _sc_segsum_node(tab0, tab1, src_p, dst_p)
            z0, z1, stz = _tc_step(sg0, sg1, bi0, bi1, deg, a, c,
                                   p['conv_W'][step],
                                   p['conv_b'][step].reshape(1, -1))
            a, c = _affine(stz, p['bn2_g'][step], p['bn2_b'][step], N)
            tab0, tab1 = z0, z1
            zs.append((z0, z1))
            affs.append((a, c))
        r_raw, st3 = _tc_concat_matmul(zs, affs, kw)
        a3, c3 = _affine(st3, p['bn3_g'], p['bn3_b'], N)
        if block < MAX_BLOCK - 1:
            bi0, bi1 = _tc_block_input(r_raw, a3, c3, ip0, ip1)

    emb = _tc_node_emb(r_raw, a3, c3, p['out_W'], p['out_b'].reshape(1, -1))
    t_raw, st4 = _tc_tanh_matmul_stats(emb, p['att_w1_W'],
                                       p['att_w1_b'].reshape(1, -1), N)
    a4, c4 = _affine(st4, p['bn4_g'], p['bn4_b'], N)
    w2f = jnp.pad(a4.reshape(-1, 1) * p['att_w2_W'],
                  ((0, 0), (0, HALF - MULTI_H)))
    b2f = jnp.pad((c4.reshape(-1) @ p['att_w2_W'] + p['att_w2_b']).reshape(1, -1),
                  ((0, 0), (0, HALF - MULTI_H)))
    a2_raw, st5 = _tc_matmul_stats(t_raw, w2f, b2f, N)
    a5, c5 = _affine(st5[:, :MULTI_H], p['bn5_g'], p['bn5_b'], N)
    a5 = jnp.pad(a5, ((0, 0), (0, HALF - MULTI_H)))
    c5 = jnp.pad(c5, ((0, 0), (0, HALF - MULTI_H)))

    g1h = (graph_ids[:, None] == jnp.arange(G)[None, :]).astype(jnp.float32)
    g1h = jnp.pad(g1h, ((0, NPAD - N), (0, 0)))
    gh = _tc_pool(a2_raw, a5, c5, g1h, emb)
    h = gh.reshape(G, MULTI_H * LATENT)

    flat = jnp.concatenate([
        p['w_n2l_W'].reshape(-1), p['w_e2l_W'].reshape(-1),
        p['conv_W'].reshape(-1), p['k_weight'].reshape(-1),
        p['out_W'].reshape(-1), p['att_w1_W'].reshape(-1),
        p['att_w2_W'].reshape(-1)]).reshape(-1, HALF)
    reg = _tc_l1(flat).reshape(())

    return (h, reg)
